# baseline (speedup 1.0000x reference)
import functools

import jax
import jax.numpy as jnp
from jax.experimental import pallas as pl
from jax.experimental.pallas import tpu as pltpu

HP = 128


def _net_kernel(node_ref, edge_ref, srcc_ref, dstc_ref, dstr_ref,
                ehw_ref, ehb_ref, eew_ref, eeb_ref,
                w4_ref, b4_ref, wc_ref, bc_ref,
                gh_ref, bth_ref, ge_ref, bte_ref,
                m0w_ref, m0b_ref, m1w_ref, m1b_ref, m2w_ref, m2b_ref,
                o_ref,
                h_scr, e_scr, soh_scr, doh_scr, doht_scr,
                *, hp, n_nodes, n_edges, n_layers):
    dot = lambda a, b: jnp.dot(a, b, preferred_element_type=jnp.float32)
    bf = jnp.bfloat16

    col_en = jax.lax.broadcasted_iota(jnp.int32, (n_edges, n_nodes), 1)
    soh_scr[...] = (col_en == srcc_ref[...]).astype(bf)
    doh_scr[...] = (col_en == dstc_ref[...]).astype(bf)
    row_ne = jax.lax.broadcasted_iota(jnp.int32, (n_nodes, n_edges), 0)
    doht_scr[...] = (row_ne == dstr_ref[...]).astype(bf)
    h_scr[...] = dot(node_ref[...], ehw_ref[...]) + ehb_ref[...]
    e_scr[...] = edge_ref[...] * eew_ref[...] + eeb_ref[...]

    zpad = jnp.zeros((hp, hp), jnp.float32)
    e2 = n_edges // 2

    def layer(l):
        h = h_scr[...]
        e = e_scr[...]

        proj = dot(h, w4_ref[l]) + b4_ref[l]
        pbf = proj.astype(bf)
        Ce = dot(e, jnp.concatenate([wc_ref[l], zpad], axis=1))[:, 0:hp]
        Ce = Ce + bc_ref[l]

        es, aggs = [], []
        for k in range(2):
            sl = pl.ds(k * e2, e2)
            DB = dot(soh_scr[sl, :], pbf[:, 0:2 * hp])
            AE = dot(doh_scr[sl, :], pbf[:, 2 * hp:4 * hp])
            e_new_k = (DB[:, 0:hp] + AE[:, hp:2 * hp]
                       + Ce[k * e2:(k + 1) * e2, :])
            sigma = jax.nn.sigmoid(e_new_k)
            msg = jnp.concatenate([sigma * DB[:, hp:2 * hp], sigma], axis=1)
            aggs.append(dot(doht_scr[:, sl], msg.astype(bf)))
            es.append(e_new_k)

        agg = aggs[0] + aggs[1]
        e_new = jnp.concatenate(es, axis=0)
        h_new = (proj[:, 2 * hp:3 * hp]
                 + agg[:, 0:hp] / (agg[:, hp:2 * hp] + 1e-6))

        def bn(x, gamma, beta):
            mu = jnp.mean(x, axis=0, keepdims=True)
            xc = x - mu
            var = jnp.mean(xc * xc, axis=0, keepdims=True)
            return xc * jax.lax.rsqrt(var + 1e-5) * gamma + beta

        h_new = jnp.maximum(bn(h_new, gh_ref[l], bth_ref[l]), 0.0)
        e_new = jnp.maximum(bn(e_new, ge_ref[l], bte_ref[l]), 0.0)

        h_scr[...] = h + h_new
        e_scr[...] = e + e_new

    def two_layers(i, _):
        layer(2 * i)
        layer(2 * i + 1)
        return None

    jax.lax.fori_loop(0, n_layers // 2, two_layers, None)

    y = jnp.maximum(dot(h_scr[...], m0w_ref[...]) + m0b_ref[...], 0.0)
    y = jnp.maximum(dot(y, m1w_ref[...]) + m1b_ref[...], 0.0)
    y = dot(y, m2w_ref[...]) + m2b_ref[...]
    n = jnp.sqrt(jnp.sum(y * y, axis=1, keepdims=True))
    o_ref[...] = y / jnp.maximum(n, 1e-12)


@jax.jit
def _forward(node_h, edge_h, src, dst, params):
    N = node_h.shape[0]
    E = edge_h.shape[0]
    L = params["w4"].shape[0]
    hp = params["w4"].shape[1]

    node_p = jnp.pad(node_h, ((0, 0), (0, hp - node_h.shape[1])))
    srcc = src.astype(jnp.int32).reshape(E, 1)
    dstc = dst.astype(jnp.int32).reshape(E, 1)
    dstr = dst.astype(jnp.int32).reshape(1, E)

    vmem = pl.BlockSpec(memory_space=pltpu.MemorySpace.VMEM)
    y = pl.pallas_call(
        functools.partial(_net_kernel, hp=hp, n_nodes=N, n_edges=E,
                          n_layers=L),
        out_shape=jax.ShapeDtypeStruct((N, hp), jnp.float32),
        in_specs=[vmem] * 23,
        out_specs=vmem,
        scratch_shapes=[
            pltpu.VMEM((N, hp), jnp.float32),
            pltpu.VMEM((E, hp), jnp.float32),
            pltpu.VMEM((E, N), jnp.bfloat16),
            pltpu.VMEM((E, N), jnp.bfloat16),
            pltpu.VMEM((N, E), jnp.bfloat16),
        ],
    )(node_p, edge_h, srcc, dstc, dstr,
      params["emb_h_w"], params["emb_h_b"], params["emb_e_w"],
      params["emb_e_b"],
      params["w4"], params["b4"], params["wc"], params["bc"],
      params["bn_h_g"], params["bn_h_b"], params["bn_e_g"], params["bn_e_b"],
      params["mlp_w0"], params["mlp_b0"], params["mlp_w1"], params["mlp_b1"],
      params["mlp_w2"], params["mlp_b2"])
    return y[:, :4]


def kernel(node_h, edge_h, src, dst,
           emb_h_w, emb_h_b, emb_e_w, emb_e_b,
           w4, b4, wc, bc,
           bn_h_g, bn_h_b, bn_e_g, bn_e_b,
           mlp_w0, mlp_b0, mlp_w1, mlp_b1, mlp_w2, mlp_b2):
    params = {
        "emb_h_w": emb_h_w, "emb_h_b": emb_h_b,
        "emb_e_w": emb_e_w, "emb_e_b": emb_e_b,
        "w4": w4, "b4": b4, "wc": wc, "bc": bc,
        "bn_h_g": bn_h_g, "bn_h_b": bn_h_b,
        "bn_e_g": bn_e_g, "bn_e_b": bn_e_b,
        "mlp_w0": mlp_w0, "mlp_b0": mlp_b0,
        "mlp_w1": mlp_w1, "mlp_b1": mlp_b1,
        "mlp_w2": mlp_w2, "mlp_b2": mlp_b2,
    }
    return _forward(node_h, edge_h, src, dst, params)


# speedup vs baseline: 1.1750x; 1.1750x over previous
import functools

import jax
import jax.numpy as jnp
from jax.experimental import pallas as pl
from jax.experimental.pallas import tpu as pltpu

HP = 128


def _net_kernel(node_ref, edge_ref, srcc_ref, dstc_ref, dstr_ref,
                ehw_ref, ehb_ref, eew_ref, eeb_ref,
                w4_ref, b4_ref, wc_ref, bc_ref,
                gh_ref, bth_ref, ge_ref, bte_ref,
                m0w_ref, m0b_ref, m1w_ref, m1b_ref, m2w_ref, m2b_ref,
                o_ref,
                h_scr, e_scr, soh_scr, doh_scr, doht_scr,
                *, hp, n_nodes, n_edges, n_layers):
    dot = lambda a, b: jnp.dot(a, b, preferred_element_type=jnp.float32)
    bf = jnp.bfloat16

    col_en = jax.lax.broadcasted_iota(jnp.int32, (n_edges, n_nodes), 1)
    soh_scr[...] = (col_en == srcc_ref[...]).astype(jnp.float32)
    doh_scr[...] = (col_en == dstc_ref[...]).astype(jnp.float32)
    row_ne = jax.lax.broadcasted_iota(jnp.int32, (n_nodes, n_edges), 0)
    doht_scr[...] = (row_ne == dstr_ref[...]).astype(jnp.float32)
    h_scr[...] = dot(node_ref[...], ehw_ref[...]) + ehb_ref[...]
    e_scr[...] = edge_ref[...] * eew_ref[...] + eeb_ref[...]

    zpad = jnp.zeros((hp, hp), jnp.float32)
    e2 = n_edges // 2

    def layer(l):
        h = h_scr[...]
        e = e_scr[...]

        proj = dot(h, w4_ref[l]) + b4_ref[l]
        pbf = proj
        Ce = dot(e, jnp.concatenate([wc_ref[l], zpad], axis=1))[:, 0:hp]
        Ce = Ce + bc_ref[l]

        DB = dot(soh_scr[...], pbf[:, 0:2 * hp])
        AE = dot(doh_scr[...], pbf[:, 2 * hp:4 * hp])

        e_new = DB[:, 0:hp] + AE[:, hp:2 * hp] + Ce
        sigma = jax.nn.sigmoid(e_new)

        msg = jnp.concatenate([sigma * DB[:, hp:2 * hp], sigma], axis=1)
        agg = dot(doht_scr[...], msg)
        h_new = (proj[:, 2 * hp:3 * hp]
                 + agg[:, 0:hp] / (agg[:, hp:2 * hp] + 1e-6))

        def bn(x, gamma, beta):
            mu = jnp.mean(x, axis=0, keepdims=True)
            m2 = jnp.mean(x * x, axis=0, keepdims=True)
            var = m2 - mu * mu
            s = jax.lax.rsqrt(var + 1e-5) * gamma
            return (x - mu) * s + beta

        h_new = jnp.maximum(bn(h_new, gh_ref[l], bth_ref[l]), 0.0)
        e_new = jnp.maximum(bn(e_new, ge_ref[l], bte_ref[l]), 0.0)

        h_scr[...] = h + h_new
        e_scr[...] = e + e_new

    def two_layers(i, _):
        layer(2 * i)
        layer(2 * i + 1)
        return None

    jax.lax.fori_loop(0, n_layers // 2, two_layers, None)

    y = jnp.maximum(dot(h_scr[...], m0w_ref[...]) + m0b_ref[...], 0.0)
    y = jnp.maximum(dot(y, m1w_ref[...]) + m1b_ref[...], 0.0)
    y = dot(y, m2w_ref[...]) + m2b_ref[...]
    n = jnp.sqrt(jnp.sum(y * y, axis=1, keepdims=True))
    o_ref[...] = y / jnp.maximum(n, 1e-12)


@jax.jit
def _forward(node_h, edge_h, src, dst, params):
    N = node_h.shape[0]
    E = edge_h.shape[0]
    L = params["w4"].shape[0]
    hp = params["w4"].shape[1]

    node_p = jnp.pad(node_h, ((0, 0), (0, hp - node_h.shape[1])))
    srcc = src.astype(jnp.int32).reshape(E, 1)
    dstc = dst.astype(jnp.int32).reshape(E, 1)
    dstr = dst.astype(jnp.int32).reshape(1, E)

    vmem = pl.BlockSpec(memory_space=pltpu.MemorySpace.VMEM)
    y = pl.pallas_call(
        functools.partial(_net_kernel, hp=hp, n_nodes=N, n_edges=E,
                          n_layers=L),
        out_shape=jax.ShapeDtypeStruct((N, hp), jnp.float32),
        in_specs=[vmem] * 23,
        out_specs=vmem,
        scratch_shapes=[
            pltpu.VMEM((N, hp), jnp.float32),
            pltpu.VMEM((E, hp), jnp.float32),
            pltpu.VMEM((E, N), jnp.float32),
            pltpu.VMEM((E, N), jnp.float32),
            pltpu.VMEM((N, E), jnp.float32),
        ],
    )(node_p, edge_h, srcc, dstc, dstr,
      params["emb_h_w"], params["emb_h_b"], params["emb_e_w"],
      params["emb_e_b"],
      params["w4"], params["b4"], params["wc"], params["bc"],
      params["bn_h_g"], params["bn_h_b"], params["bn_e_g"], params["bn_e_b"],
      params["mlp_w0"], params["mlp_b0"], params["mlp_w1"], params["mlp_b1"],
      params["mlp_w2"], params["mlp_b2"])
    return y[:, :4]


def kernel(node_h, edge_h, src, dst,
           emb_h_w, emb_h_b, emb_e_w, emb_e_b,
           w4, b4, wc, bc,
           bn_h_g, bn_h_b, bn_e_g, bn_e_b,
           mlp_w0, mlp_b0, mlp_w1, mlp_b1, mlp_w2, mlp_b2):
    params = {
        "emb_h_w": emb_h_w, "emb_h_b": emb_h_b,
        "emb_e_w": emb_e_w, "emb_e_b": emb_e_b,
        "w4": w4, "b4": b4, "wc": wc, "bc": bc,
        "bn_h_g": bn_h_g, "bn_h_b": bn_h_b,
        "bn_e_g": bn_e_g, "bn_e_b": bn_e_b,
        "mlp_w0": mlp_w0, "mlp_b0": mlp_b0,
        "mlp_w1": mlp_w1, "mlp_b1": mlp_b1,
        "mlp_w2": mlp_w2, "mlp_b2": mlp_b2,
    }
    return _forward(node_h, edge_h, src, dst, params)


# speedup vs baseline: 1.1768x; 1.0015x over previous
import functools

import jax
import jax.numpy as jnp
from jax.experimental import pallas as pl
from jax.experimental.pallas import tpu as pltpu

HP = 128


def _net_kernel(node_ref, edge_ref, srcc_ref, dstc_ref, dstr_ref,
                ehw_ref, ehb_ref, eew_ref, eeb_ref,
                w4_ref, b4_ref, wc_ref, bc_ref,
                gh_ref, bth_ref, ge_ref, bte_ref,
                m0w_ref, m0b_ref, m1w_ref, m1b_ref, m2w_ref, m2b_ref,
                o_ref,
                h_scr, e_scr, soh_scr, doh_scr, doht_scr,
                *, hp, n_nodes, n_edges, n_layers):
    dot = lambda a, b: jnp.dot(a, b, preferred_element_type=jnp.float32)
    bf = jnp.bfloat16

    col_en = jax.lax.broadcasted_iota(jnp.int32, (n_edges, n_nodes), 1)
    soh_scr[...] = (col_en == srcc_ref[...]).astype(jnp.float32)
    doh_scr[...] = (col_en == dstc_ref[...]).astype(jnp.float32)
    row_ne = jax.lax.broadcasted_iota(jnp.int32, (n_nodes, n_edges), 0)
    doht_scr[...] = (row_ne == dstr_ref[...]).astype(jnp.float32)
    h_scr[...] = dot(node_ref[...], ehw_ref[...]) + ehb_ref[...]
    e_scr[...] = edge_ref[...] * eew_ref[...] + eeb_ref[...]

    zpad = jnp.zeros((hp, hp), jnp.float32)
    e2 = n_edges // 2

    def layer(l):
        h = h_scr[...]
        e = e_scr[...]

        proj = dot(h, w4_ref[l]) + b4_ref[l]
        pbf = proj
        Ce = dot(e, wc_ref[l]) + bc_ref[l]

        DB = dot(soh_scr[...], pbf[:, 0:2 * hp])
        AE = dot(doh_scr[...], pbf[:, 3 * hp:4 * hp])

        e_new = DB[:, 0:hp] + AE + Ce
        sigma = jax.nn.sigmoid(e_new)

        msg = jnp.concatenate([sigma * DB[:, hp:2 * hp], sigma], axis=1)
        agg = dot(doht_scr[...], msg)
        h_new = (proj[:, 2 * hp:3 * hp]
                 + agg[:, 0:hp] / (agg[:, hp:2 * hp] + 1e-6))

        def bn(x, gamma, beta):
            mu = jnp.mean(x, axis=0, keepdims=True)
            m2 = jnp.mean(x * x, axis=0, keepdims=True)
            var = m2 - mu * mu
            s = jax.lax.rsqrt(var + 1e-5) * gamma
            return (x - mu) * s + beta

        h_new = jnp.maximum(bn(h_new, gh_ref[l], bth_ref[l]), 0.0)
        e_new = jnp.maximum(bn(e_new, ge_ref[l], bte_ref[l]), 0.0)

        h_scr[...] = h + h_new
        e_scr[...] = e + e_new

    def two_layers(i, _):
        layer(2 * i)
        layer(2 * i + 1)
        return None

    jax.lax.fori_loop(0, n_layers // 2, two_layers, None)

    y = jnp.maximum(dot(h_scr[...], m0w_ref[...]) + m0b_ref[...], 0.0)
    y = jnp.maximum(dot(y, m1w_ref[...]) + m1b_ref[...], 0.0)
    y = dot(y, m2w_ref[...]) + m2b_ref[...]
    n = jnp.sqrt(jnp.sum(y * y, axis=1, keepdims=True))
    o_ref[...] = y / jnp.maximum(n, 1e-12)


@jax.jit
def _forward(node_h, edge_h, src, dst, params):
    N = node_h.shape[0]
    E = edge_h.shape[0]
    L = params["w4"].shape[0]
    hp = params["w4"].shape[1]

    node_p = jnp.pad(node_h, ((0, 0), (0, hp - node_h.shape[1])))
    srcc = src.astype(jnp.int32).reshape(E, 1)
    dstc = dst.astype(jnp.int32).reshape(E, 1)
    dstr = dst.astype(jnp.int32).reshape(1, E)

    vmem = pl.BlockSpec(memory_space=pltpu.MemorySpace.VMEM)
    y = pl.pallas_call(
        functools.partial(_net_kernel, hp=hp, n_nodes=N, n_edges=E,
                          n_layers=L),
        out_shape=jax.ShapeDtypeStruct((N, hp), jnp.float32),
        in_specs=[vmem] * 23,
        out_specs=vmem,
        scratch_shapes=[
            pltpu.VMEM((N, hp), jnp.float32),
            pltpu.VMEM((E, hp), jnp.float32),
            pltpu.VMEM((E, N), jnp.float32),
            pltpu.VMEM((E, N), jnp.float32),
            pltpu.VMEM((N, E), jnp.float32),
        ],
    )(node_p, edge_h, srcc, dstc, dstr,
      params["emb_h_w"], params["emb_h_b"], params["emb_e_w"],
      params["emb_e_b"],
      params["w4"], params["b4"], params["wc"], params["bc"],
      params["bn_h_g"], params["bn_h_b"], params["bn_e_g"], params["bn_e_b"],
      params["mlp_w0"], params["mlp_b0"], params["mlp_w1"], params["mlp_b1"],
      params["mlp_w2"], params["mlp_b2"])
    return y[:, :4]


def kernel(node_h, edge_h, src, dst,
           emb_h_w, emb_h_b, emb_e_w, emb_e_b,
           w4, b4, wc, bc,
           bn_h_g, bn_h_b, bn_e_g, bn_e_b,
           mlp_w0, mlp_b0, mlp_w1, mlp_b1, mlp_w2, mlp_b2):
    params = {
        "emb_h_w": emb_h_w, "emb_h_b": emb_h_b,
        "emb_e_w": emb_e_w, "emb_e_b": emb_e_b,
        "w4": w4, "b4": b4, "wc": wc, "bc": bc,
        "bn_h_g": bn_h_g, "bn_h_b": bn_h_b,
        "bn_e_g": bn_e_g, "bn_e_b": bn_e_b,
        "mlp_w0": mlp_w0, "mlp_b0": mlp_b0,
        "mlp_w1": mlp_w1, "mlp_b1": mlp_b1,
        "mlp_w2": mlp_w2, "mlp_b2": mlp_b2,
    }
    return _forward(node_h, edge_h, src, dst, params)
